# Initial kernel scaffold; baseline (speedup 1.0000x reference)
import jax
import jax.numpy as jnp
from jax.experimental import pallas as pl


def kernel(input, coords):
    raise NotImplementedError("write your pallas kernel here")


# speedup vs baseline: 1.8650x; 1.8650x over previous
import functools

import jax
import jax.numpy as jnp
from jax import lax
from jax.experimental import pallas as pl
from jax.experimental.pallas import tpu as pltpu
from jax.experimental.pallas import tpu_sc as plsc

N = 1048576
C = 64
M = 262144

NC = 2
NS = 16
L = 16

HALF = 26240
PAD = 16
ACC_ROWS = HALF + PAD
SLAB = NC * HALF
NPASS = 5
LAST_HALF = (M - (NPASS - 1) * SLAB) // NC

CHUNK = N // NS
BLK = 512
RING = 1024
TCH = 128
ZROWS = 64


def _kernel_body(inp, coords, out, cblk, pidx_r, lrow_r, pbuf, rbuf, rowsbuf,
                 zbuf, acc, sem):
    c = lax.axis_index("c")
    s = lax.axis_index("s")
    iota = lax.iota(jnp.int32, L)
    zvec = jnp.zeros((L,), jnp.float32)

    def _zb(r, _):
        for k in range(C // L):
            zbuf[r, pl.ds(k * L, L)] = zvec
        return 0

    lax.fori_loop(0, ZROWS, _zb, 0)

    def drain(tpos, tdone, limit):
        def cond(tdone):
            return tdone + limit <= tpos

        def body(tdone):
            off = tdone & (RING - 1)
            for t in range(TCH // L):
                pbuf[pl.ds(t * L, L)] = pidx_r[pl.ds(off + t * L, L)]
                rbuf[pl.ds(t * L, L)] = lrow_r[pl.ds(off + t * L, L)]
            pltpu.async_copy(inp.at[pbuf], rowsbuf, sem).wait()
            pltpu.sync_copy(rowsbuf, acc.at[rbuf], add=True)
            return tdone + TCH

        return lax.while_loop(cond, body, tdone)

    def run_pass(p, half_p):
        pass_base = p * SLAB
        lo = pass_base + c * half_p
        hi = lo + half_p

        zstripe = ACC_ROWS // NS
        for k in range(zstripe // ZROWS):
            pltpu.sync_copy(zbuf,
                            acc.at[pl.ds(s * zstripe + k * ZROWS, ZROWS)])
        ztail = zstripe % ZROWS
        if ztail:
            pltpu.sync_copy(
                zbuf.at[pl.ds(0, ztail)],
                acc.at[pl.ds(s * zstripe + (zstripe // ZROWS) * ZROWS, ztail)])
        plsc.subcore_barrier()

        chunk_base = s * CHUNK

        def block(b, carry):
            tpos, tdone = carry
            base = chunk_base + b * BLK
            pltpu.sync_copy(coords.at[pl.ds(base, BLK)], cblk)

            def _filter(i, tpos):
                v = cblk[pl.ds(i * L, L)]
                m = (v >= lo) & (v < hi)
                mi = m.astype(jnp.int32)
                cum = plsc.cumsum(mi)
                pos = (tpos + cum - mi) & (RING - 1)
                plsc.store_scatter(lrow_r, [pos], v - lo, mask=m)
                plsc.store_scatter(pidx_r, [pos], base + i * L + iota, mask=m)
                return tpos + jnp.sum(mi)

            tpos = lax.fori_loop(0, BLK // L, _filter, tpos)
            tdone = drain(tpos, tdone, TCH)
            return tpos, tdone

        tpos, tdone = lax.fori_loop(0, CHUNK // BLK, block,
                                    (jnp.int32(0), jnp.int32(0)))

        for t in range(TCH // L):
            ppos = (tpos + t * L + iota) & (RING - 1)
            plsc.store_scatter(pidx_r, [ppos], t * L + iota)
            plsc.store_scatter(lrow_r, [ppos], HALF + iota)
        drain(tpos + TCH - 1, tdone, TCH)

        plsc.subcore_barrier()

        share = half_p // NS
        pltpu.sync_copy(
            acc.at[pl.ds(s * share, share)],
            out.at[pl.ds(pass_base + c * half_p + s * share, share)])
        plsc.subcore_barrier()

    for p in range(NPASS - 1):
        run_pass(p, HALF)
    run_pass(NPASS - 1, LAST_HALF)


@jax.jit
def kernel(input, coords):
    coords = coords.astype(jnp.int32)
    mesh = plsc.VectorSubcoreMesh(core_axis_name="c", subcore_axis_name="s")
    f = functools.partial(
        pl.kernel,
        out_type=jax.ShapeDtypeStruct((M, C), jnp.float32),
        mesh=mesh,
        compiler_params=pltpu.CompilerParams(
            needs_layout_passes=False, use_tc_tiling_on_sc=False),
        scratch_types=[
            pltpu.VMEM((BLK,), jnp.int32),
            pltpu.VMEM((RING,), jnp.int32),
            pltpu.VMEM((RING,), jnp.int32),
            pltpu.VMEM((TCH,), jnp.int32),
            pltpu.VMEM((TCH,), jnp.int32),
            pltpu.VMEM((TCH, C), jnp.float32),
            pltpu.VMEM((ZROWS, C), jnp.float32),
            pltpu.VMEM_SHARED((ACC_ROWS, C), jnp.float32),
            pltpu.SemaphoreType.DMA,
        ],
    )(_kernel_body)
    return f(input, coords)


# speedup vs baseline: 2.2326x; 1.1971x over previous
import functools

import jax
import jax.numpy as jnp
from jax import lax
from jax.experimental import pallas as pl
from jax.experimental.pallas import tpu as pltpu
from jax.experimental.pallas import tpu_sc as plsc

N = 1048576
C = 64
M = 262144

NC = 2
NS = 16
L = 16

HALF = 26240
PAD = 16
ACC_ROWS = HALF + PAD
SLAB = NC * HALF
NPASS = 5
LAST_HALF = (M - (NPASS - 1) * SLAB) // NC

CHUNK = N // NS
BLK = 512
RING = 1024
TCH = 128
ZROWS = 64


def _kernel_body(inp, coords, out, cblk_a, cblk_b, pidx_r, lrow_r,
                 pbuf_a, rbuf_a, pbuf_b, rbuf_b, rows_a, rows_b,
                 zbuf, acc, sem_a, sem_b, sem_ca, sem_cb):
    c = lax.axis_index("c")
    s = lax.axis_index("s")
    iota = lax.iota(jnp.int32, L)
    zvec = jnp.zeros((L,), jnp.float32)

    def _zb(r, _):
        for k in range(C // L):
            zbuf[r, pl.ds(k * L, L)] = zvec
        return 0

    lax.fori_loop(0, ZROWS, _zb, 0)

    def xfer_start(tdone, pbuf, rbuf, rows, sem):
        off = tdone & (RING - 1)
        for t in range(TCH // L):
            pbuf[pl.ds(t * L, L)] = pidx_r[pl.ds(off + t * L, L)]
            rbuf[pl.ds(t * L, L)] = lrow_r[pl.ds(off + t * L, L)]
        return pltpu.async_copy(inp.at[pbuf], rows, sem)

    def drain(tpos, tdone):
        def cond2(td):
            return td + 2 * TCH <= tpos

        def body2(td):
            da = xfer_start(td, pbuf_a, rbuf_a, rows_a, sem_a)
            db = xfer_start(td + TCH, pbuf_b, rbuf_b, rows_b, sem_b)
            da.wait()
            pltpu.sync_copy(rows_a, acc.at[rbuf_a], add=True)
            db.wait()
            pltpu.sync_copy(rows_b, acc.at[rbuf_b], add=True)
            return td + 2 * TCH

        tdone = lax.while_loop(cond2, body2, tdone)

        def cond1(td):
            return td + TCH <= tpos

        def body1(td):
            xfer_start(td, pbuf_a, rbuf_a, rows_a, sem_a).wait()
            pltpu.sync_copy(rows_a, acc.at[rbuf_a], add=True)
            return td + TCH

        return lax.while_loop(cond1, body1, tdone)

    def run_pass(p, half_p):
        pass_base = p * SLAB
        lo = pass_base + c * half_p
        hi = lo + half_p

        zstripe = ACC_ROWS // NS
        for k in range(zstripe // ZROWS):
            pltpu.sync_copy(zbuf,
                            acc.at[pl.ds(s * zstripe + k * ZROWS, ZROWS)])
        ztail = zstripe % ZROWS
        if ztail:
            pltpu.sync_copy(
                zbuf.at[pl.ds(0, ztail)],
                acc.at[pl.ds(s * zstripe + (zstripe // ZROWS) * ZROWS, ztail)])
        plsc.subcore_barrier()

        chunk_base = s * CHUNK

        def filter_block(cblk, base, tpos, tdone):
            def _filter(i, tpos_v):
                v = cblk[pl.ds(i * L, L)]
                m = (v >= lo) & (v < hi)
                mi = m.astype(jnp.int32)
                cum = plsc.cumsum(mi)
                pos = (tpos_v + cum - mi) & (RING - 1)
                plsc.store_scatter(lrow_r, [pos], v - lo, mask=m)
                plsc.store_scatter(pidx_r, [pos], base + i * L + iota, mask=m)
                return tpos_v + plsc.all_reduce_population_count(m)

            tpos_v = lax.fori_loop(0, BLK // L, _filter,
                                   jnp.full((L,), tpos, jnp.int32))
            tpos = jnp.max(tpos_v)
            tdone = drain(tpos, tdone)
            return tpos, tdone

        pltpu.async_copy(coords.at[pl.ds(chunk_base, BLK)], cblk_a, sem_ca)

        def block_pair(b2, carry):
            tpos, tdone = carry
            base_a = chunk_base + (2 * b2) * BLK
            base_b = base_a + BLK
            pltpu.async_copy(coords.at[pl.ds(base_b, BLK)], cblk_b, sem_cb)
            pltpu.make_async_copy(coords.at[pl.ds(0, BLK)], cblk_a,
                                  sem_ca).wait()
            tpos, tdone = filter_block(cblk_a, base_a, tpos, tdone)
            next_a = jnp.minimum(base_b + BLK, N - BLK)
            pltpu.async_copy(coords.at[pl.ds(next_a, BLK)], cblk_a, sem_ca)
            pltpu.make_async_copy(coords.at[pl.ds(0, BLK)], cblk_b,
                                  sem_cb).wait()
            tpos, tdone = filter_block(cblk_b, base_b, tpos, tdone)
            return tpos, tdone

        tpos, tdone = lax.fori_loop(0, CHUNK // BLK // 2, block_pair,
                                    (jnp.int32(0), jnp.int32(0)))
        pltpu.make_async_copy(coords.at[pl.ds(0, BLK)], cblk_a, sem_ca).wait()

        for t in range(TCH // L):
            ppos = (tpos + t * L + iota) & (RING - 1)
            plsc.store_scatter(pidx_r, [ppos], t * L + iota)
            plsc.store_scatter(lrow_r, [ppos], HALF + iota)
        drain(tpos + TCH - 1, tdone)

        plsc.subcore_barrier()

        share = half_p // NS
        pltpu.sync_copy(
            acc.at[pl.ds(s * share, share)],
            out.at[pl.ds(pass_base + c * half_p + s * share, share)])
        plsc.subcore_barrier()

    for p in range(NPASS - 1):
        run_pass(p, HALF)
    run_pass(NPASS - 1, LAST_HALF)


@jax.jit
def kernel(input, coords):
    coords = coords.astype(jnp.int32)
    mesh = plsc.VectorSubcoreMesh(core_axis_name="c", subcore_axis_name="s")
    f = functools.partial(
        pl.kernel,
        out_type=jax.ShapeDtypeStruct((M, C), jnp.float32),
        mesh=mesh,
        compiler_params=pltpu.CompilerParams(
            needs_layout_passes=False, use_tc_tiling_on_sc=False),
        scratch_types=[
            pltpu.VMEM((BLK,), jnp.int32),
            pltpu.VMEM((BLK,), jnp.int32),
            pltpu.VMEM((RING,), jnp.int32),
            pltpu.VMEM((RING,), jnp.int32),
            pltpu.VMEM((TCH,), jnp.int32),
            pltpu.VMEM((TCH,), jnp.int32),
            pltpu.VMEM((TCH,), jnp.int32),
            pltpu.VMEM((TCH,), jnp.int32),
            pltpu.VMEM((TCH, C), jnp.float32),
            pltpu.VMEM((TCH, C), jnp.float32),
            pltpu.VMEM((ZROWS, C), jnp.float32),
            pltpu.VMEM_SHARED((ACC_ROWS, C), jnp.float32),
            pltpu.SemaphoreType.DMA,
            pltpu.SemaphoreType.DMA,
            pltpu.SemaphoreType.DMA,
            pltpu.SemaphoreType.DMA,
        ],
    )(_kernel_body)
    return f(input, coords)
